# baseline (speedup 1.0000x reference)
import functools

import jax
import jax.numpy as jnp
from jax import lax
from jax.experimental import pallas as pl
from jax.experimental.pallas import tpu as pltpu
from jax.experimental.pallas import tpu_sc as plsc

N = 100000
E = 3200000
G = 1024
K = 16

NUM_CORES = 2
NUM_SUBCORES = 16
NW = NUM_CORES * NUM_SUBCORES

CH = 128
GC = 2
GPT = 392
E_PAD = NW * GPT * GC * CH
NP_PAD = 102400
DUMMY = NP_PAD
N_ACC = NP_PAD + 2048
ZROWS = N_ACC // NUM_SUBCORES
CHT_P = NP_PAD // NW // CH
G_ACC = 1152
GZ = G_ACC // NUM_SUBCORES


def _mesh():
    return plsc.VectorSubcoreMesh(core_axis_name="c", subcore_axis_name="s")


def _edge_pass(U, V, srcp, dstp, eap, wd2):
    @functools.partial(
        pl.kernel,
        mesh=_mesh(),
        compiler_params=pltpu.CompilerParams(use_tc_tiling_on_sc=False),
        out_type=jax.ShapeDtypeStruct((NUM_CORES, NP_PAD, K), jnp.float32),
        scratch_types=[
            pltpu.VMEM((3, GC, CH), jnp.int32),
            pltpu.VMEM((3, GC, CH), jnp.int32),
            pltpu.VMEM((3, GC, CH), jnp.float32),
            pltpu.VMEM((2, GC, CH, K), jnp.float32),
            pltpu.VMEM((2, GC, CH, K), jnp.float32),
            pltpu.VMEM((K,), jnp.float32),
            pltpu.VMEM_SHARED((N_ACC, K), jnp.float32),
            pltpu.SemaphoreType.DMA,
            pltpu.SemaphoreType.DMA,
            pltpu.SemaphoreType.DMA,
        ],
    )
    def k(u_hbm, v_hbm, src_hbm, dst_hbm, ea_hbm, wd_hbm, out_hbm,
          srcg, dstg, eag, ub, vb, wdb, acc, sem_ig, sem_ga, sem_sc):
        cid = lax.axis_index("c")
        sid = lax.axis_index("s")
        wid = cid * NUM_SUBCORES + sid

        def zrow(i, _):
            ub[0, 0, i, :] = jnp.zeros((K,), jnp.float32)
            return 0
        lax.fori_loop(0, CH, zrow, 0)
        base = sid * ZROWS

        def zchunk(j, _):
            pltpu.sync_copy(ub.at[0, 0], acc.at[pl.ds(base + j * CH, CH)])
            return 0
        lax.fori_loop(0, ZROWS // CH, zchunk, 0)
        plsc.subcore_barrier()

        pltpu.sync_copy(wd_hbm, wdb)
        wd2v = wdb[:]
        one = jnp.float32(1.0)
        two = jnp.float32(2.0)

        def idx_issue(g):
            gg = wid * GPT + g
            s = lax.rem(g, 3)
            pltpu.async_copy(src_hbm.at[gg], srcg.at[s], sem_ig)
            pltpu.async_copy(dst_hbm.at[gg], dstg.at[s], sem_ig)
            pltpu.async_copy(ea_hbm.at[gg], eag.at[s], sem_ig)

        def idx_wait(g):
            gg = wid * GPT + g
            s = lax.rem(g, 3)
            pltpu.make_async_copy(src_hbm.at[gg], srcg.at[s], sem_ig).wait()
            pltpu.make_async_copy(dst_hbm.at[gg], dstg.at[s], sem_ig).wait()
            pltpu.make_async_copy(ea_hbm.at[gg], eag.at[s], sem_ig).wait()

        def gath_issue(g):
            s = lax.rem(g, 3)
            p = lax.rem(g, 2)
            for j in range(GC):
                pltpu.async_copy(u_hbm.at[srcg.at[s, j]], ub.at[p, j], sem_ga)
                pltpu.async_copy(v_hbm.at[dstg.at[s, j]], vb.at[p, j], sem_ga)

        def gath_wait(g):
            s = lax.rem(g, 3)
            p = lax.rem(g, 2)
            for j in range(GC):
                pltpu.make_async_copy(
                    u_hbm.at[srcg.at[s, j]], ub.at[p, j], sem_ga).wait()
                pltpu.make_async_copy(
                    v_hbm.at[dstg.at[s, j]], vb.at[p, j], sem_ga).wait()

        def scat_issue(g):
            s = lax.rem(g, 3)
            p = lax.rem(g, 2)
            for j in range(GC):
                pltpu.async_copy(ub.at[p, j], acc.at[dstg.at[s, j]], sem_sc,
                                 add=True)

        def scat_wait(g):
            s = lax.rem(g, 3)
            p = lax.rem(g, 2)
            for j in range(GC):
                pltpu.make_async_copy(
                    ub.at[p, j], acc.at[dstg.at[s, j]], sem_sc).wait()

        def compute(g):
            s = lax.rem(g, 3)
            p = lax.rem(g, 2)
            for j in range(GC):
                def qblk(q, _):
                    av = eag[s, j, pl.ds(q * K, K)]
                    for i in range(K):
                        e = q * K + i
                        z = ub[p, j, e, :] + vb[p, j, e, :] + av[i] * wd2v
                        ub[p, j, e, :] = one - two / (jnp.exp(z) + one)
                    return 0
                lax.fori_loop(0, CH // K, qblk, 0)

        idx_issue(0)
        idx_issue(1)
        idx_wait(0)
        gath_issue(0)

        def body(g, _):
            @pl.when(g > 0)
            def _():
                scat_wait(g - 1)

            @pl.when(g < GPT - 1)
            def _():
                idx_wait(g + 1)

            @pl.when(g < GPT - 2)
            def _():
                idx_issue(g + 2)
            gath_wait(g)

            @pl.when(g < GPT - 1)
            def _():
                gath_issue(g + 1)
            compute(g)
            scat_issue(g)
            return 0
        lax.fori_loop(0, GPT, body, 0)
        scat_wait(GPT - 1)
        plsc.subcore_barrier()

        @pl.when(sid == 0)
        def _():
            pltpu.sync_copy(acc.at[pl.ds(0, NP_PAD)], out_hbm.at[cid])

    return k(U, V, srcp, dstp, eap, wd2)


def _pool(parts, batchp):
    @functools.partial(
        pl.kernel,
        mesh=_mesh(),
        compiler_params=pltpu.CompilerParams(use_tc_tiling_on_sc=False),
        out_type=(
            jax.ShapeDtypeStruct((NUM_CORES, G, K), jnp.float32),
            jax.ShapeDtypeStruct((NUM_CORES, G, K), jnp.float32),
        ),
        scratch_types=[
            pltpu.VMEM((CH,), jnp.int32),
            pltpu.VMEM((CH, K), jnp.float32),
            pltpu.VMEM((CH, K), jnp.float32),
            pltpu.VMEM((CH, K), jnp.float32),
            pltpu.VMEM_SHARED((G_ACC, K), jnp.float32),
            pltpu.VMEM_SHARED((G_ACC, K), jnp.float32),
        ],
    )
    def k(parts_hbm, batch_hbm, rsum_hbm, cnt_hbm,
          bb, h0, h1, onesb, rs, cs):
        cid = lax.axis_index("c")
        sid = lax.axis_index("s")
        wid = cid * NUM_SUBCORES + sid

        def fill(i, _):
            h0[i, :] = jnp.zeros((K,), jnp.float32)
            onesb[i, :] = jnp.ones((K,), jnp.float32)
            return 0
        lax.fori_loop(0, CH, fill, 0)
        base = sid * GZ
        pltpu.sync_copy(h0.at[pl.ds(0, GZ)], rs.at[pl.ds(base, GZ)])
        pltpu.sync_copy(h0.at[pl.ds(0, GZ)], cs.at[pl.ds(base, GZ)])
        plsc.subcore_barrier()

        def chunk(ch, _):
            g = wid * CHT_P + ch
            pltpu.sync_copy(batch_hbm.at[g], bb)
            pltpu.sync_copy(parts_hbm.at[0, pl.ds(g * CH, CH)], h0)
            pltpu.sync_copy(parts_hbm.at[1, pl.ds(g * CH, CH)], h1)

            def row(e, _):
                h0[e, :] = h0[e, :] + h1[e, :]
                return 0
            lax.fori_loop(0, CH, row, 0)
            pltpu.sync_copy(h0, rs.at[bb], add=True)
            pltpu.sync_copy(onesb, cs.at[bb], add=True)
            return 0
        lax.fori_loop(0, CHT_P, chunk, 0)
        plsc.subcore_barrier()

        @pl.when(sid == 0)
        def _():
            pltpu.sync_copy(rs.at[pl.ds(0, G)], rsum_hbm.at[cid])
            pltpu.sync_copy(cs.at[pl.ds(0, G)], cnt_hbm.at[cid])

    return k(parts, batchp)


_R = 1000


def _proj0(xp, Wu, Wv, bv):
    def body(xp_ref, wu_ref, wv_ref, bv_ref, u_ref, v_ref):
        xpb = xp_ref[...]
        u_ref[...] = jnp.dot(xpb, wu_ref[...], preferred_element_type=jnp.float32)
        v_ref[...] = (jnp.dot(xpb, wv_ref[...], preferred_element_type=jnp.float32)
                      + bv_ref[...])

    return pl.pallas_call(
        body,
        grid=(N // _R,),
        in_specs=[
            pl.BlockSpec((_R, 8), lambda i: (i, 0)),
            pl.BlockSpec((8, K), lambda i: (0, 0)),
            pl.BlockSpec((8, K), lambda i: (0, 0)),
            pl.BlockSpec((1, K), lambda i: (0, 0)),
        ],
        out_specs=[
            pl.BlockSpec((_R, K), lambda i: (i, 0)),
            pl.BlockSpec((_R, K), lambda i: (i, 0)),
        ],
        out_shape=[
            jax.ShapeDtypeStruct((N, K), jnp.float32),
            jax.ShapeDtypeStruct((N, K), jnp.float32),
        ],
    )(xp, Wu, Wv, bv)


def _proj1(parts, p, Wua, Wuc, Wva, Wvc, bv):
    def body(pa_ref, pb_ref, p_ref, wua_ref, wuc_ref, wva_ref, wvc_ref,
             bv_ref, u_ref, v_ref):
        h = pa_ref[0] + pb_ref[0]
        pb = p_ref[...]
        u_ref[...] = (jnp.dot(h, wua_ref[...], preferred_element_type=jnp.float32)
                      + jnp.dot(pb, wuc_ref[...], preferred_element_type=jnp.float32))
        v_ref[...] = (jnp.dot(h, wva_ref[...], preferred_element_type=jnp.float32)
                      + jnp.dot(pb, wvc_ref[...], preferred_element_type=jnp.float32)
                      + bv_ref[...])

    return pl.pallas_call(
        body,
        grid=(N // _R,),
        in_specs=[
            pl.BlockSpec((1, _R, K), lambda i: (0, i, 0)),
            pl.BlockSpec((1, _R, K), lambda i: (1, i, 0)),
            pl.BlockSpec((_R, 3), lambda i: (i, 0)),
            pl.BlockSpec((K, K), lambda i: (0, 0)),
            pl.BlockSpec((3, K), lambda i: (0, 0)),
            pl.BlockSpec((K, K), lambda i: (0, 0)),
            pl.BlockSpec((3, K), lambda i: (0, 0)),
            pl.BlockSpec((1, K), lambda i: (0, 0)),
        ],
        out_specs=[
            pl.BlockSpec((_R, K), lambda i: (i, 0)),
            pl.BlockSpec((_R, K), lambda i: (i, 0)),
        ],
        out_shape=[
            jax.ShapeDtypeStruct((N, K), jnp.float32),
            jax.ShapeDtypeStruct((N, K), jnp.float32),
        ],
    )(parts, parts, p, Wua, Wuc, Wva, Wvc, bv)


def _final(rsum, cnt, Wp, bp):
    def body(rs_ref, cn_ref, wp_ref, bp_ref, pred_ref, rep_ref):
        rs = rs_ref[0] + rs_ref[1]
        c = cn_ref[0, :, 0:1] + cn_ref[1, :, 0:1]
        rep = rs / jnp.maximum(c, 1.0)
        rep_ref[...] = rep
        pred_ref[...] = (jnp.dot(rep, wp_ref[...], preferred_element_type=jnp.float32)
                         + bp_ref[...])

    return pl.pallas_call(
        body,
        out_shape=[
            jax.ShapeDtypeStruct((G, 1), jnp.float32),
            jax.ShapeDtypeStruct((G, K), jnp.float32),
        ],
    )(rsum, cnt, Wp, bp)


def kernel(x, p, edge_index, edge_attr, batch, W0, b0, W1, b1, Wp, bp):
    src = edge_index[0]
    dst = edge_index[1]
    pad = E_PAD - E
    srcp = jnp.concatenate([src, jnp.zeros((pad,), jnp.int32)]
                           ).reshape(-1, GC, CH)
    dstp = jnp.concatenate([dst, jnp.full((pad,), DUMMY, jnp.int32)]
                           ).reshape(-1, GC, CH)
    eap = jnp.concatenate([edge_attr[:, 0], jnp.zeros((pad,), jnp.float32)]
                          ).reshape(-1, GC, CH)
    batchp = jnp.concatenate([batch, jnp.full((NP_PAD - N,), G, jnp.int32)]
                             ).reshape(-1, CH)

    Wa0, Wb0, Wc0, wd0 = W0[0:5], W0[5:10], W0[10:13], W0[13]
    Wu0 = 2.0 * jnp.concatenate([Wa0, -Wc0], axis=0)
    Wv0 = 2.0 * jnp.concatenate([Wb0, Wc0], axis=0)
    bv0 = (2.0 * b0).reshape(1, K)
    wd20 = 2.0 * wd0
    xp = jnp.concatenate([x, p], axis=1)

    U0, V0 = _proj0(xp, Wu0, Wv0, bv0)
    parts0 = _edge_pass(U0, V0, srcp, dstp, eap, wd20)

    Wa1, Wb1, Wc1, wd1 = W1[0:16], W1[16:32], W1[32:35], W1[35]
    U1, V1 = _proj1(parts0, p, 2.0 * Wa1, -2.0 * Wc1, 2.0 * Wb1, 2.0 * Wc1,
                    (2.0 * b1).reshape(1, K))
    parts1 = _edge_pass(U1, V1, srcp, dstp, eap, 2.0 * wd1)

    rsum, cnt = _pool(parts1, batchp)
    pred, rep = _final(rsum, cnt, Wp, bp.reshape(1, 1))
    return (pred, rep)


# speedup vs baseline: 3.0376x; 3.0376x over previous
import functools

import jax
import jax.numpy as jnp
from jax import lax
from jax.experimental import pallas as pl
from jax.experimental.pallas import tpu as pltpu
from jax.experimental.pallas import tpu_sc as plsc

N = 100000
E = 3200000
G = 1024
K = 16

NUM_CORES = 2
NUM_SUBCORES = 16
NW = NUM_CORES * NUM_SUBCORES

CH = 128
GC = 2
GPT = 392
E_PAD = NW * GPT * GC * CH
NP_PAD = 102400
DUMMY = NP_PAD
N_ACC = NP_PAD + 128
ZROWS = N_ACC // NUM_SUBCORES
CHT_P = NP_PAD // NW // CH
G_ACC = 1152
GZ = G_ACC // NUM_SUBCORES


def _mesh():
    return plsc.VectorSubcoreMesh(core_axis_name="c", subcore_axis_name="s")


def _edge_pass(U, V, srcp, dstp, eap, wd2):
    @functools.partial(
        pl.kernel,
        mesh=_mesh(),
        compiler_params=pltpu.CompilerParams(use_tc_tiling_on_sc=False),
        out_type=jax.ShapeDtypeStruct((NUM_CORES, NP_PAD, K), jnp.float32),
        scratch_types=[
            pltpu.VMEM((3, GC, CH), jnp.int32),
            pltpu.VMEM((3, GC, CH), jnp.int32),
            pltpu.VMEM((3, GC, CH), jnp.float32),
            pltpu.VMEM((2, GC, CH, K), jnp.float32),
            pltpu.VMEM((2, GC, CH, K), jnp.float32),
            pltpu.VMEM((2, GC, CH, K), jnp.float32),
            pltpu.VMEM((K,), jnp.float32),
            pltpu.VMEM_SHARED((N_ACC, K), jnp.float32),
            pltpu.SemaphoreType.DMA,
            pltpu.SemaphoreType.DMA,
            pltpu.SemaphoreType.DMA,
        ],
    )
    def k(u_hbm, v_hbm, src_hbm, dst_hbm, ea_hbm, wd_hbm, out_hbm,
          srcg, dstg, eag, ub, vb, sb, wdb, acc, sem_ig, sem_ga, sem_sc):
        cid = lax.axis_index("c")
        sid = lax.axis_index("s")
        wid = cid * NUM_SUBCORES + sid

        def zrow(i, _):
            ub[0, 0, i, :] = jnp.zeros((K,), jnp.float32)
            return 0
        lax.fori_loop(0, CH, zrow, 0)
        base = sid * ZROWS

        def zchunk(j, _):
            pltpu.sync_copy(ub.at[0, 0], acc.at[pl.ds(base + j * CH, CH)])
            return 0
        lax.fori_loop(0, ZROWS // CH, zchunk, 0)
        pltpu.sync_copy(ub.at[0, 0, pl.ds(0, ZROWS % CH)],
                        acc.at[pl.ds(base + (ZROWS // CH) * CH, ZROWS % CH)])
        plsc.subcore_barrier()

        pltpu.sync_copy(wd_hbm, wdb)
        wd2v = wdb[:]
        one = jnp.float32(1.0)
        two = jnp.float32(2.0)
        lanes = [jnp.full((K, 1), i, jnp.int32) for i in range(K)]
        _dnums = lax.GatherDimensionNumbers(
            offset_dims=(), collapsed_slice_dims=(0,), start_index_map=(0,))

        def _bcast(av, i):
            return lax.gather(av, lanes[i], _dnums, (1,),
                              mode=lax.GatherScatterMode.PROMISE_IN_BOUNDS)

        def idx_issue(g):
            gg = wid * GPT + g
            s = lax.rem(g, 3)
            pltpu.async_copy(src_hbm.at[gg], srcg.at[s], sem_ig)
            pltpu.async_copy(dst_hbm.at[gg], dstg.at[s], sem_ig)
            pltpu.async_copy(ea_hbm.at[gg], eag.at[s], sem_ig)

        def idx_wait(g):
            gg = wid * GPT + g
            s = lax.rem(g, 3)
            pltpu.make_async_copy(src_hbm.at[gg], srcg.at[s], sem_ig).wait()
            pltpu.make_async_copy(dst_hbm.at[gg], dstg.at[s], sem_ig).wait()
            pltpu.make_async_copy(ea_hbm.at[gg], eag.at[s], sem_ig).wait()

        def gath_issue(g, p):
            s = lax.rem(g, 3)
            for j in range(GC):
                pltpu.async_copy(u_hbm.at[srcg.at[s, j]], ub.at[p, j], sem_ga)
                pltpu.async_copy(v_hbm.at[dstg.at[s, j]], vb.at[p, j], sem_ga)

        def gath_wait(g, p):
            s = lax.rem(g, 3)
            for j in range(GC):
                pltpu.make_async_copy(
                    u_hbm.at[srcg.at[s, j]], ub.at[p, j], sem_ga).wait()
                pltpu.make_async_copy(
                    v_hbm.at[dstg.at[s, j]], vb.at[p, j], sem_ga).wait()

        def scat_issue(g, p):
            s = lax.rem(g, 3)
            for j in range(GC):
                pltpu.async_copy(sb.at[p, j], acc.at[dstg.at[s, j]], sem_sc,
                                 add=True)

        def scat_wait(g, p):
            s = lax.rem(g, 3)
            for j in range(GC):
                pltpu.make_async_copy(
                    sb.at[p, j], acc.at[dstg.at[s, j]], sem_sc).wait()

        def compute(g, p):
            s = lax.rem(g, 3)
            for j in range(GC):
                def qblk(q, _):
                    av = eag[s, j, pl.ds(q * K, K)]
                    for i in range(K):
                        e = q * K + i
                        ai = _bcast(av, i)
                        z = ub[p, j, e, :] + vb[p, j, e, :] + ai * wd2v
                        sb[p, j, e, :] = one - two / (jnp.exp(z) + one)
                    return 0
                lax.fori_loop(0, CH // K, qblk, 0)

        def step(g, p):
            @pl.when(g > 0)
            def _():
                scat_wait(g - 1, 1 - p)

            @pl.when(g < GPT - 1)
            def _():
                idx_wait(g + 1)

            @pl.when(g < GPT - 2)
            def _():
                idx_issue(g + 2)
            gath_wait(g, p)

            @pl.when(g < GPT - 1)
            def _():
                gath_issue(g + 1, 1 - p)
            compute(g, p)
            scat_issue(g, p)

        idx_issue(0)
        idx_issue(1)
        idx_wait(0)
        gath_issue(0, 0)

        def body(t, _):
            step(2 * t, 0)
            step(2 * t + 1, 1)
            return 0
        lax.fori_loop(0, GPT // 2, body, 0)
        scat_wait(GPT - 1, 1)
        plsc.subcore_barrier()

        @pl.when(sid == 0)
        def _():
            pltpu.sync_copy(acc.at[pl.ds(0, NP_PAD)], out_hbm.at[cid])

    return k(U, V, srcp, dstp, eap, wd2)


def _pool(parts, batchp):
    @functools.partial(
        pl.kernel,
        mesh=_mesh(),
        compiler_params=pltpu.CompilerParams(use_tc_tiling_on_sc=False),
        out_type=(
            jax.ShapeDtypeStruct((NUM_CORES, G, K), jnp.float32),
            jax.ShapeDtypeStruct((NUM_CORES, G, K), jnp.float32),
        ),
        scratch_types=[
            pltpu.VMEM((CH,), jnp.int32),
            pltpu.VMEM((CH, K), jnp.float32),
            pltpu.VMEM((CH, K), jnp.float32),
            pltpu.VMEM((CH, K), jnp.float32),
            pltpu.VMEM_SHARED((G_ACC, K), jnp.float32),
            pltpu.VMEM_SHARED((G_ACC, K), jnp.float32),
        ],
    )
    def k(parts_hbm, batch_hbm, rsum_hbm, cnt_hbm,
          bb, h0, h1, onesb, rs, cs):
        cid = lax.axis_index("c")
        sid = lax.axis_index("s")
        wid = cid * NUM_SUBCORES + sid

        def fill(i, _):
            h0[i, :] = jnp.zeros((K,), jnp.float32)
            onesb[i, :] = jnp.ones((K,), jnp.float32)
            return 0
        lax.fori_loop(0, CH, fill, 0)
        base = sid * GZ
        pltpu.sync_copy(h0.at[pl.ds(0, GZ)], rs.at[pl.ds(base, GZ)])
        pltpu.sync_copy(h0.at[pl.ds(0, GZ)], cs.at[pl.ds(base, GZ)])
        plsc.subcore_barrier()

        def chunk(ch, _):
            g = wid * CHT_P + ch
            pltpu.sync_copy(batch_hbm.at[g], bb)
            pltpu.sync_copy(parts_hbm.at[0, pl.ds(g * CH, CH)], h0)
            pltpu.sync_copy(parts_hbm.at[1, pl.ds(g * CH, CH)], h1)

            def row(e, _):
                h0[e, :] = h0[e, :] + h1[e, :]
                return 0
            lax.fori_loop(0, CH, row, 0)
            pltpu.sync_copy(h0, rs.at[bb], add=True)
            pltpu.sync_copy(onesb, cs.at[bb], add=True)
            return 0
        lax.fori_loop(0, CHT_P, chunk, 0)
        plsc.subcore_barrier()

        @pl.when(sid == 0)
        def _():
            pltpu.sync_copy(rs.at[pl.ds(0, G)], rsum_hbm.at[cid])
            pltpu.sync_copy(cs.at[pl.ds(0, G)], cnt_hbm.at[cid])

    return k(parts, batchp)


_R = 1000


def _proj0(xp, Wu, Wv, bv):
    def body(xp_ref, wu_ref, wv_ref, bv_ref, u_ref, v_ref):
        xpb = xp_ref[...]
        u_ref[...] = jnp.dot(xpb, wu_ref[...], preferred_element_type=jnp.float32)
        v_ref[...] = (jnp.dot(xpb, wv_ref[...], preferred_element_type=jnp.float32)
                      + bv_ref[...])

    return pl.pallas_call(
        body,
        grid=(N // _R,),
        in_specs=[
            pl.BlockSpec((_R, 8), lambda i: (i, 0)),
            pl.BlockSpec((8, K), lambda i: (0, 0)),
            pl.BlockSpec((8, K), lambda i: (0, 0)),
            pl.BlockSpec((1, K), lambda i: (0, 0)),
        ],
        out_specs=[
            pl.BlockSpec((_R, K), lambda i: (i, 0)),
            pl.BlockSpec((_R, K), lambda i: (i, 0)),
        ],
        out_shape=[
            jax.ShapeDtypeStruct((N, K), jnp.float32),
            jax.ShapeDtypeStruct((N, K), jnp.float32),
        ],
    )(xp, Wu, Wv, bv)


def _proj1(parts, p, Wua, Wuc, Wva, Wvc, bv):
    def body(pa_ref, pb_ref, p_ref, wua_ref, wuc_ref, wva_ref, wvc_ref,
             bv_ref, u_ref, v_ref):
        h = pa_ref[0] + pb_ref[0]
        pb = p_ref[...]
        u_ref[...] = (jnp.dot(h, wua_ref[...], preferred_element_type=jnp.float32)
                      + jnp.dot(pb, wuc_ref[...], preferred_element_type=jnp.float32))
        v_ref[...] = (jnp.dot(h, wva_ref[...], preferred_element_type=jnp.float32)
                      + jnp.dot(pb, wvc_ref[...], preferred_element_type=jnp.float32)
                      + bv_ref[...])

    return pl.pallas_call(
        body,
        grid=(N // _R,),
        in_specs=[
            pl.BlockSpec((1, _R, K), lambda i: (0, i, 0)),
            pl.BlockSpec((1, _R, K), lambda i: (1, i, 0)),
            pl.BlockSpec((_R, 3), lambda i: (i, 0)),
            pl.BlockSpec((K, K), lambda i: (0, 0)),
            pl.BlockSpec((3, K), lambda i: (0, 0)),
            pl.BlockSpec((K, K), lambda i: (0, 0)),
            pl.BlockSpec((3, K), lambda i: (0, 0)),
            pl.BlockSpec((1, K), lambda i: (0, 0)),
        ],
        out_specs=[
            pl.BlockSpec((_R, K), lambda i: (i, 0)),
            pl.BlockSpec((_R, K), lambda i: (i, 0)),
        ],
        out_shape=[
            jax.ShapeDtypeStruct((N, K), jnp.float32),
            jax.ShapeDtypeStruct((N, K), jnp.float32),
        ],
    )(parts, parts, p, Wua, Wuc, Wva, Wvc, bv)


def _final(rsum, cnt, Wp, bp):
    def body(rs_ref, cn_ref, wp_ref, bp_ref, pred_ref, rep_ref):
        rs = rs_ref[0] + rs_ref[1]
        c = cn_ref[0, :, 0:1] + cn_ref[1, :, 0:1]
        rep = rs / jnp.maximum(c, 1.0)
        rep_ref[...] = rep
        pred_ref[...] = (jnp.dot(rep, wp_ref[...], preferred_element_type=jnp.float32)
                         + bp_ref[...])

    return pl.pallas_call(
        body,
        out_shape=[
            jax.ShapeDtypeStruct((G, 1), jnp.float32),
            jax.ShapeDtypeStruct((G, K), jnp.float32),
        ],
    )(rsum, cnt, Wp, bp)


def kernel(x, p, edge_index, edge_attr, batch, W0, b0, W1, b1, Wp, bp):
    src = edge_index[0]
    dst = edge_index[1]
    pad = E_PAD - E
    srcp = jnp.concatenate([src, jnp.zeros((pad,), jnp.int32)]
                           ).reshape(-1, GC, CH)
    dstp = jnp.concatenate([dst, jnp.full((pad,), DUMMY, jnp.int32)]
                           ).reshape(-1, GC, CH)
    eap = jnp.concatenate([edge_attr[:, 0], jnp.zeros((pad,), jnp.float32)]
                          ).reshape(-1, GC, CH)
    batchp = jnp.concatenate([batch, jnp.full((NP_PAD - N,), G, jnp.int32)]
                             ).reshape(-1, CH)

    Wa0, Wb0, Wc0, wd0 = W0[0:5], W0[5:10], W0[10:13], W0[13]
    Wu0 = 2.0 * jnp.concatenate([Wa0, -Wc0], axis=0)
    Wv0 = 2.0 * jnp.concatenate([Wb0, Wc0], axis=0)
    bv0 = (2.0 * b0).reshape(1, K)
    wd20 = 2.0 * wd0
    xp = jnp.concatenate([x, p], axis=1)

    U0, V0 = _proj0(xp, Wu0, Wv0, bv0)
    parts0 = _edge_pass(U0, V0, srcp, dstp, eap, wd20)

    Wa1, Wb1, Wc1, wd1 = W1[0:16], W1[16:32], W1[32:35], W1[35]
    U1, V1 = _proj1(parts0, p, 2.0 * Wa1, -2.0 * Wc1, 2.0 * Wb1, 2.0 * Wc1,
                    (2.0 * b1).reshape(1, K))
    parts1 = _edge_pass(U1, V1, srcp, dstp, eap, 2.0 * wd1)

    rsum, cnt = _pool(parts1, batchp)
    pred, rep = _final(rsum, cnt, Wp, bp.reshape(1, 1))
    return (pred, rep)
